# Initial kernel scaffold; baseline (speedup 1.0000x reference)
import jax
import jax.numpy as jnp
from jax.experimental import pallas as pl


def kernel(x, edge_index, W_l1, W_r1, b1, W_l2, W_r2, b2, W_out, b_out):
    raise NotImplementedError("write your pallas kernel here")


# speedup vs baseline: 4.9363x; 4.9363x over previous
import functools

import jax
import jax.numpy as jnp
from jax import lax
from jax.experimental import pallas as pl
from jax.experimental.pallas import tpu as pltpu
from jax.experimental.pallas import tpu_sc as plsc

N = 10000
E = 320000
D = 128
H = 128
C = 64

NC = 2
NS = 16
NW = NC * NS
CH = 128
NCH = 79
E_PAD = NW * NCH * CH
ROWS_PER_TILE = 640
N_PAD = NS * ROWS_PER_TILE
ROWB = 400


def _sc_aggregate(with_counts: bool):
  mesh = plsc.VectorSubcoreMesh(core_axis_name="c", subcore_axis_name="s")
  out_type = [jax.ShapeDtypeStruct((NC, N_PAD, H), jnp.float32)]
  scratch = [
      pltpu.VMEM((NCH, CH), jnp.int32),
      pltpu.VMEM((NCH, CH), jnp.int32),
      pltpu.VMEM((CH, H), jnp.float32),
      pltpu.VMEM((16, H), jnp.float32),
      pltpu.VMEM_SHARED((N_PAD, H), jnp.float32),
      pltpu.SemaphoreType.DMA,
  ]
  if with_counts:
    out_type.append(jax.ShapeDtypeStruct((NW, N_PAD), jnp.float32))
    scratch.append(pltpu.VMEM((N_PAD,), jnp.float32))

  @functools.partial(
      pl.kernel, out_type=out_type, scratch_types=scratch, mesh=mesh,
      name="sage_sc_aggregate",
      compiler_params=pltpu.CompilerParams(needs_layout_passes=False),
  )
  def body(src_hbm, dst_hbm, y_hbm, agg_hbm, *rest):
    if with_counts:
      cnt_hbm, src_v, dst_v, rows_v, zb_v, acc_sh, sem, hist_v = rest
    else:
      src_v, dst_v, rows_v, zb_v, acc_sh, sem = rest
    c = lax.axis_index("c")
    s = lax.axis_index("s")
    wid = c * NS + s

    pltpu.sync_copy(src_hbm.at[wid], src_v)
    pltpu.sync_copy(dst_hbm.at[wid], dst_v)

    zeros16 = jnp.zeros((16,), jnp.float32)
    for i in range(16):
      for j in range(H // 16):
        zb_v[i, pl.ds(j * 16, 16)] = zeros16

    base = s * ROWS_PER_TILE
    for r in range(ROWS_PER_TILE // 16):
      pltpu.sync_copy(zb_v, acc_sh.at[pl.ds(base + r * 16, 16)])

    if with_counts:
      def zh(r, carry):
        hist_v[pl.ds(r * 16, 16)] = zeros16
        return carry
      lax.fori_loop(0, N_PAD // 16, zh, 0)

    plsc.subcore_barrier()

    ones16 = jnp.full((16,), 1.0, jnp.float32)

    def chunk(j, carry):
      pltpu.async_copy(y_hbm.at[src_v.at[j]], rows_v, sem).wait()
      pltpu.sync_copy(rows_v, acc_sh.at[dst_v.at[j]], add=True)
      if with_counts:
        for k in range(CH // 16):
          idx16 = dst_v[j, pl.ds(k * 16, 16)]
          plsc.addupdate_scatter(hist_v, [idx16], ones16)
      return carry
    lax.fori_loop(0, NCH, chunk, 0)

    plsc.subcore_barrier()

    pltpu.sync_copy(acc_sh.at[pl.ds(base, ROWS_PER_TILE)],
                    agg_hbm.at[c, pl.ds(base, ROWS_PER_TILE)])
    if with_counts:
      pltpu.sync_copy(hist_v, cnt_hbm.at[wid])

  return body


_sc_agg_counts = _sc_aggregate(with_counts=True)
_sc_agg = _sc_aggregate(with_counts=False)


def _tc_pre(x, W_l, W_r, b):
  def body(x_ref, wl_ref, wr_ref, b_ref, y_ref, r_ref):
    xb = x_ref[...]
    y_ref[...] = jnp.dot(xb, wl_ref[...], preferred_element_type=jnp.float32)
    r_ref[...] = jnp.dot(xb, wr_ref[...],
                         preferred_element_type=jnp.float32) + b_ref[...]

  grid = (N // ROWB,)
  return pl.pallas_call(
      body,
      grid=grid,
      in_specs=[
          pl.BlockSpec((ROWB, D), lambda i: (i, 0)),
          pl.BlockSpec((D, H), lambda i: (0, 0)),
          pl.BlockSpec((D, H), lambda i: (0, 0)),
          pl.BlockSpec((1, H), lambda i: (0, 0)),
      ],
      out_specs=[
          pl.BlockSpec((ROWB, H), lambda i: (i, 0)),
          pl.BlockSpec((ROWB, H), lambda i: (i, 0)),
      ],
      out_shape=[
          jax.ShapeDtypeStruct((N, H), jnp.float32),
          jax.ShapeDtypeStruct((N, H), jnp.float32),
      ],
  )(x, W_l, W_r, b.reshape(1, H))


def _tc_mid(agg, cnt, r, W_l, W_r, b):
  def body(agg_ref, cnt_ref, r_ref, wl_ref, wr_ref, b_ref, y_ref, r2_ref):
    ssum = agg_ref[0] + agg_ref[1]
    deg = jnp.maximum(jnp.sum(cnt_ref[...], axis=1), 1.0)
    h = jnp.maximum(ssum / deg[:, None] + r_ref[...], 0.0)
    y_ref[...] = jnp.dot(h, wl_ref[...], preferred_element_type=jnp.float32)
    r2_ref[...] = jnp.dot(h, wr_ref[...],
                          preferred_element_type=jnp.float32) + b_ref[...]

  grid = (N // ROWB,)
  return pl.pallas_call(
      body,
      grid=grid,
      in_specs=[
          pl.BlockSpec((NC, ROWB, H), lambda i: (0, i, 0)),
          pl.BlockSpec((ROWB, NW), lambda i: (i, 0)),
          pl.BlockSpec((ROWB, H), lambda i: (i, 0)),
          pl.BlockSpec((H, H), lambda i: (0, 0)),
          pl.BlockSpec((H, H), lambda i: (0, 0)),
          pl.BlockSpec((1, H), lambda i: (0, 0)),
      ],
      out_specs=[
          pl.BlockSpec((ROWB, H), lambda i: (i, 0)),
          pl.BlockSpec((ROWB, H), lambda i: (i, 0)),
      ],
      out_shape=[
          jax.ShapeDtypeStruct((N, H), jnp.float32),
          jax.ShapeDtypeStruct((N, H), jnp.float32),
      ],
  )(agg, cnt, r, W_l, W_r, b.reshape(1, H))


def _tc_out(agg, cnt, r, W_out, b_out):
  def body(agg_ref, cnt_ref, r_ref, w_ref, b_ref, o_ref):
    ssum = agg_ref[0] + agg_ref[1]
    deg = jnp.maximum(jnp.sum(cnt_ref[...], axis=1), 1.0)
    h = jnp.maximum(ssum / deg[:, None] + r_ref[...], 0.0)
    o_ref[...] = jnp.dot(h, w_ref[...],
                         preferred_element_type=jnp.float32) + b_ref[...]

  grid = (N // ROWB,)
  return pl.pallas_call(
      body,
      grid=grid,
      in_specs=[
          pl.BlockSpec((NC, ROWB, H), lambda i: (0, i, 0)),
          pl.BlockSpec((ROWB, NW), lambda i: (i, 0)),
          pl.BlockSpec((ROWB, H), lambda i: (i, 0)),
          pl.BlockSpec((H, C), lambda i: (0, 0)),
          pl.BlockSpec((1, C), lambda i: (0, 0)),
      ],
      out_specs=pl.BlockSpec((ROWB, C), lambda i: (i, 0)),
      out_shape=jax.ShapeDtypeStruct((N, C), jnp.float32),
  )(agg, cnt, r, W_out, b_out.reshape(1, C))


def kernel(x, edge_index, W_l1, W_r1, b1, W_l2, W_r2, b2, W_out, b_out):
  src = edge_index[0].astype(jnp.int32)
  dst = edge_index[1].astype(jnp.int32)
  pad = E_PAD - E
  src_p = jnp.concatenate([src, jnp.zeros((pad,), jnp.int32)])
  dst_p = jnp.concatenate([dst, jnp.full((pad,), N, jnp.int32)])
  src_p = src_p.reshape(NW, NCH, CH)
  dst_p = dst_p.reshape(NW, NCH, CH)

  y1, r1 = _tc_pre(x, W_l1, W_r1, b1)
  agg1, cnt = _sc_agg_counts(src_p, dst_p, y1)
  agg1 = agg1[:, :N, :]
  cnt = cnt[:, :N].T

  y2, r2 = _tc_mid(agg1, cnt, r1, W_l2, W_r2, b2)
  (agg2,) = _sc_agg(src_p, dst_p, y2)
  agg2 = agg2[:, :N, :]

  return _tc_out(agg2, cnt, r2, W_out, b_out)
